# baseline (speedup 1.0000x reference)
import functools

import jax
import jax.numpy as jnp
from jax import lax
from jax.experimental import pallas as pl
from jax.experimental.pallas import tpu as pltpu
from jax.experimental.pallas import tpu_sc as plsc

D_MODEL = 64
LANES = 16
SCALE = 8.0
CHUNK = 512


def _emb_body(idx_hbm, table_hbm, out_hbm, idx_v, rows_v, sem, *, nc, per_w):
    wid = lax.axis_index("s") * nc + lax.axis_index("c")
    base = wid * per_w
    pltpu.sync_copy(idx_hbm.at[pl.ds(base, per_w)], idx_v)

    nchunks = per_w // CHUNK

    def chunk_body(ci, carry):
        off = ci * CHUNK
        pltpu.async_copy(
            table_hbm.at[idx_v.at[pl.ds(off, CHUNK)]], rows_v, sem
        ).wait()

        def row_body(r, c2):
            for j in range(D_MODEL // LANES):
                sl = pl.ds(j * LANES, LANES)
                rows_v[r, sl] = rows_v[r, sl] * SCALE
            return c2

        lax.fori_loop(0, CHUNK, row_body, 0, unroll=2)

        pltpu.sync_copy(rows_v, out_hbm.at[pl.ds(base + off, CHUNK)])
        return carry

    lax.fori_loop(0, nchunks, chunk_body, 0)


def kernel(x, table):
    b, s = x.shape
    n = b * s
    idx = x.reshape(n).astype(jnp.int32)

    info = plsc.get_sparse_core_info()
    nc, ns = info.num_cores, info.num_subcores
    nw = nc * ns
    per_w = n // nw

    mesh = plsc.VectorSubcoreMesh(core_axis_name="c", subcore_axis_name="s")
    emb = functools.partial(
        pl.kernel,
        mesh=mesh,
        out_type=jax.ShapeDtypeStruct((n, D_MODEL), jnp.float32),
        scratch_types=[
            pltpu.VMEM((per_w,), jnp.int32),
            pltpu.VMEM((CHUNK, D_MODEL), jnp.float32),
            pltpu.SemaphoreType.DMA,
        ],
        compiler_params=pltpu.CompilerParams(use_tc_tiling_on_sc=False),
    )(functools.partial(_emb_body, nc=nc, per_w=per_w))

    out = emb(idx, table)
    return out.reshape(b, s, D_MODEL)
